# baseline (speedup 1.0000x reference)
import jax
import jax.numpy as jnp
from jax import lax
from jax.experimental import pallas as pl
from jax.experimental.pallas import tpu as pltpu
from jax.experimental.pallas import tpu_sc as plsc

NC = 2
NS = 16
NW = NC * NS
L = 16
TILE = 128


def _sc_partials(h_t, pos_t, batch_idx, n, d, b):
    n_sc = n - n % TILE
    chunk = (n_sc + NW - 1) // NW
    chunk = (chunk + TILE - 1) // TILE * TILE
    rest = n_sc - (NW - 1) * chunk
    assert rest > 0 and rest % TILE == 0 and (4 * b) % NS == 0
    groups_full = chunk // L
    groups_rest = rest // L
    half_a = (chunk // 2 + TILE - 1) // TILE * TILE
    half_b = chunk - half_a
    assert half_b > 0 and (rest // 2) % TILE == 0
    stripe = TILE
    nstripes = 4 * b // stripe
    assert nstripes <= NS

    mesh = plsc.VectorSubcoreMesh(
        core_axis_name="c", subcore_axis_name="s", num_cores=NC, num_subcores=NS
    )

    @pl.kernel(
        out_type=jax.ShapeDtypeStruct((NC, 4 * b), jnp.float32),
        mesh=mesh,
        scratch_types=[
            pltpu.VMEM((8, chunk), jnp.float32),
            pltpu.VMEM((3, chunk), jnp.float32),
            pltpu.VMEM((chunk,), jnp.int32),
            pltpu.VMEM((4 * b,), jnp.float32),
            pltpu.VMEM((NS, stripe), jnp.float32),
            pltpu.VMEM((stripe,), jnp.float32),
            pltpu.VMEM_SHARED((NS, 4 * b), jnp.float32),
            pltpu.SemaphoreType.DMA,
            pltpu.SemaphoreType.DMA,
        ],
        compiler_params=pltpu.CompilerParams(
            needs_layout_passes=False,
            skip_device_barrier=True,
            disable_bounds_checks=True,
            disable_semaphore_checks=True,
        ),
    )
    def sc_kernel(h_hbm, pos_hbm, idx_hbm, out_hbm,
                  h_v, pos_v, idx_v, acc_v, red_v, str_v, sp, semA, semB):
        c = lax.axis_index("c")
        s = lax.axis_index("s")
        wid = c * NS + s
        base = wid * chunk
        is_last = wid == NW - 1

        szA = jnp.where(is_last, rest // 2, half_a)
        szA = pl.multiple_of(szA, TILE)
        szB = jnp.where(is_last, rest // 2, half_b)
        szB = pl.multiple_of(szB, TILE)
        dA = [
            pltpu.make_async_copy(
                h_hbm.at[pl.ds(0, 8), pl.ds(base, szA)],
                h_v.at[:, pl.ds(0, szA)], semA),
            pltpu.make_async_copy(
                pos_hbm.at[:, pl.ds(base, szA)],
                pos_v.at[:, pl.ds(0, szA)], semA),
            pltpu.make_async_copy(
                idx_hbm.at[pl.ds(base, szA)], idx_v.at[pl.ds(0, szA)], semA),
        ]
        for d in dA:
            d.start()
        baseB = base + szA
        baseB = pl.multiple_of(baseB, TILE)
        dB = [
            pltpu.make_async_copy(
                h_hbm.at[pl.ds(0, 8), pl.ds(baseB, szB)],
                h_v.at[:, pl.ds(half_a, szB)], semB),
            pltpu.make_async_copy(
                pos_hbm.at[:, pl.ds(baseB, szB)],
                pos_v.at[:, pl.ds(half_a, szB)], semB),
            pltpu.make_async_copy(
                idx_hbm.at[pl.ds(baseB, szB)],
                idx_v.at[pl.ds(half_a, szB)], semB),
        ]
        for d in dB:
            d.start()

        limA = szA
        limB = half_a + szB

        zeros = jnp.zeros((L,), jnp.float32)
        for i in range(4 * b // L):
            acc_v[pl.ds(i * L, L)] = zeros

        ones = jnp.ones((L,), jnp.float32)

        iota = lax.iota(jnp.int32, L)
        r3v = iota * 0 + 3
        r4v = r3v + 1
        r5v = r3v + 2
        r6v = r3v + 3
        p0v = iota * 0
        p1v = p0v + 1
        p2v = p0v + 2

        def make_body(off, stride):
            step = iota * stride + off

            def body(g, carry):
                rowv = step + g
                bidx = plsc.load_gather(idx_v, [rowv])
                c0 = plsc.load_gather(h_v, [r3v, rowv])
                c1 = plsc.load_gather(h_v, [r4v, rowv])
                c2 = plsc.load_gather(h_v, [r5v, rowv])
                c3 = plsc.load_gather(h_v, [r6v, rowv])
                cond = (c1 > c0) & (c1 >= c2) & (c1 >= c3)
                px = plsc.load_gather(pos_v, [p0v, rowv])
                py = plsc.load_gather(pos_v, [p1v, rowv])
                pz = plsc.load_gather(pos_v, [p2v, rowv])
                plsc.addupdate_scatter(acc_v, [bidx], ones, mask=cond)
                plsc.addupdate_scatter(acc_v, [bidx + b], px, mask=cond)
                plsc.addupdate_scatter(acc_v, [bidx + 2 * b], py, mask=cond)
                plsc.addupdate_scatter(acc_v, [bidx + 3 * b], pz, mask=cond)
                return carry

            return body

        sA = szA // L
        sB = szB // L
        for d in dA:
            d.wait()
        lax.fori_loop(0, sA, make_body(0, sA), 0)
        for d in dB:
            d.wait()
        lax.fori_loop(0, sB, make_body(half_a, sB), 0)

        pltpu.sync_copy(acc_v, sp.at[s])
        plsc.subcore_barrier()

        @pl.when(s < nstripes)
        def _():
            col0 = s * stripe
            pltpu.sync_copy(sp.at[:, pl.ds(col0, stripe)], red_v)
            for k in range(stripe // L):
                t = red_v[0, pl.ds(k * L, L)]
                for i in range(1, NS):
                    t = t + red_v[i, pl.ds(k * L, L)]
                str_v[pl.ds(k * L, L)] = t
            pltpu.sync_copy(str_v, out_hbm.at[c, pl.ds(col0, stripe)])

    return sc_kernel(h_t, pos_t, batch_idx)


def _tc_combine(partials, h_t, pos_t, idx_tail, n, d, b, tail):
    blk = (n - tail) // TILE

    def body(p_ref, ht_ref, pt3_ref, it_ref, pt_ref, pd_ref):
        s = jnp.sum(p_ref[...], axis=0, keepdims=True)

        ht = ht_ref[...]
        c0 = ht[3:4, :]
        c1 = ht[4:5, :]
        c2 = ht[5:6, :]
        c3 = ht[6:7, :]
        w = ((c1 > c0) & (c1 >= c2) & (c1 >= c3)).astype(jnp.float32)
        pos3 = pt3_ref[...]
        vals = jnp.concatenate([w, pos3 * w], axis=0)
        valsk = jax.lax.slice(vals, (0, 0), (4, tail))
        seg = jax.lax.broadcasted_iota(jnp.int32, (b, tail), 0)
        oh = (seg == it_ref[...][None, :]).astype(jnp.float32)
        corr = jax.lax.dot_general(
            valsk, oh, (((1,), (1,)), ((), ())),
            precision=jax.lax.Precision.HIGHEST,
            preferred_element_type=jnp.float32,
        )

        cnt = s[:, 0:b] + corr[0:1]
        sx = s[:, b:2 * b] + corr[1:2]
        sy = s[:, 2 * b:3 * b] + corr[2:3]
        sz = s[:, 3 * b:4 * b] + corr[3:4]
        c = jnp.maximum(cnt, 1.0)
        mx, my, mz = sx / c, sy / c, sz / c
        pt = jnp.sqrt(mx * mx + my * my + mz * mz)
        pt_ref[...] = pt
        pd_ref[...] = jnp.concatenate([mx / pt, my / pt, mz / pt], axis=0)

    return pl.pallas_call(
        body,
        grid=(1,),
        in_specs=[
            pl.BlockSpec(partials.shape, lambda i: (0, 0)),
            pl.BlockSpec((d, TILE), lambda i: (0, blk)),
            pl.BlockSpec((3, TILE), lambda i: (0, blk)),
            pl.BlockSpec((tail,), lambda i: (0,)),
        ],
        out_specs=[
            pl.BlockSpec((1, b), lambda i: (0, 0)),
            pl.BlockSpec((3, b), lambda i: (0, 0)),
        ],
        out_shape=[
            jax.ShapeDtypeStruct((1, b), jnp.float32),
            jax.ShapeDtypeStruct((3, b), jnp.float32),
        ],
    )(partials, h_t, pos_t, idx_tail)


def kernel(x_global_features, h, pos_pxpypz_at_vertex, batch_idx):
    n, d = h.shape
    b = x_global_features.shape[0]
    tail = n % TILE
    h_t = h.T
    pos_t = pos_pxpypz_at_vertex.T
    idx_tail = lax.slice(batch_idx, (n - tail,), (n,))
    partials = _sc_partials(h_t, pos_t, batch_idx, n, d, b)
    pt, pd = _tc_combine(partials, h_t, pos_t, idx_tail, n, d, b, tail)
    return pt.reshape(b), pd.T


# speedup vs baseline: 1.1702x; 1.1702x over previous
import jax
import jax.numpy as jnp
from jax import lax
from jax.experimental import pallas as pl
from jax.experimental.pallas import tpu as pltpu
from jax.experimental.pallas import tpu_sc as plsc

NC = 2
NS = 16
NW = NC * NS
L = 16
TILE = 128


def _sc_partials(h_t, pos_t, batch_idx, n, d, b):
    n_sc = n - n % TILE
    chunk = (n_sc + NW - 1) // NW
    chunk = (chunk + TILE - 1) // TILE * TILE
    rest = n_sc - (NW - 1) * chunk
    assert rest > 0 and rest % TILE == 0 and (4 * b) % NS == 0
    groups_full = chunk // L
    groups_rest = rest // L
    half_a = (chunk // 2 + TILE - 1) // TILE * TILE
    half_b = chunk - half_a
    assert half_b > 0 and (rest // 2) % TILE == 0
    stripe = TILE
    nstripes = 4 * b // stripe
    assert nstripes <= NS

    mesh = plsc.VectorSubcoreMesh(
        core_axis_name="c", subcore_axis_name="s", num_cores=NC, num_subcores=NS
    )

    @pl.kernel(
        out_type=jax.ShapeDtypeStruct((NC, 4 * b), jnp.float32),
        mesh=mesh,
        scratch_types=[
            pltpu.VMEM((8, chunk), jnp.float32),
            pltpu.VMEM((3, chunk), jnp.float32),
            pltpu.VMEM((chunk,), jnp.int32),
            pltpu.VMEM((4 * b,), jnp.float32),
            pltpu.VMEM((NS, stripe), jnp.float32),
            pltpu.VMEM((stripe,), jnp.float32),
            pltpu.VMEM_SHARED((NS, 4 * b), jnp.float32),
            pltpu.SemaphoreType.DMA,
            pltpu.SemaphoreType.DMA,
        ],
        compiler_params=pltpu.CompilerParams(
            needs_layout_passes=False,
            skip_device_barrier=True,
            disable_bounds_checks=True,
            disable_semaphore_checks=True,
        ),
    )
    def sc_kernel(h_hbm, pos_hbm, idx_hbm, out_hbm,
                  h_v, pos_v, idx_v, acc_v, red_v, str_v, sp, semA, semB):
        c = lax.axis_index("c")
        s = lax.axis_index("s")
        wid = c * NS + s
        base = wid * chunk
        is_last = wid == NW - 1

        szA = jnp.where(is_last, rest // 2, half_a)
        szA = pl.multiple_of(szA, TILE)
        szB = jnp.where(is_last, rest // 2, half_b)
        szB = pl.multiple_of(szB, TILE)
        dA = [
            pltpu.make_async_copy(
                h_hbm.at[pl.ds(0, 8), pl.ds(base, szA)],
                h_v.at[:, pl.ds(0, szA)], semA),
            pltpu.make_async_copy(
                pos_hbm.at[:, pl.ds(base, szA)],
                pos_v.at[:, pl.ds(0, szA)], semA),
            pltpu.make_async_copy(
                idx_hbm.at[pl.ds(base, szA)], idx_v.at[pl.ds(0, szA)], semA),
        ]
        for d in dA:
            d.start()
        baseB = base + szA
        baseB = pl.multiple_of(baseB, TILE)
        dB = [
            pltpu.make_async_copy(
                h_hbm.at[pl.ds(0, 8), pl.ds(baseB, szB)],
                h_v.at[:, pl.ds(szA, szB)], semB),
            pltpu.make_async_copy(
                pos_hbm.at[:, pl.ds(baseB, szB)],
                pos_v.at[:, pl.ds(szA, szB)], semB),
            pltpu.make_async_copy(
                idx_hbm.at[pl.ds(baseB, szB)],
                idx_v.at[pl.ds(szA, szB)], semB),
        ]
        for d in dB:
            d.start()

        gA = jnp.where(is_last, (rest // 2) // L, half_a // L)
        groups = jnp.where(is_last, groups_rest, groups_full)

        zeros = jnp.zeros((L,), jnp.float32)
        for i in range(4 * b // L):
            acc_v[pl.ds(i * L, L)] = zeros

        ones = jnp.ones((L,), jnp.float32)

        def body(g, carry):
            o = g * L
            bidx = idx_v[pl.ds(o, L)]
            c0 = h_v[3, pl.ds(o, L)]
            c1 = h_v[4, pl.ds(o, L)]
            c2 = h_v[5, pl.ds(o, L)]
            c3 = h_v[6, pl.ds(o, L)]
            cond = (c1 > c0) & (c1 >= c2) & (c1 >= c3)
            px = pos_v[0, pl.ds(o, L)]
            py = pos_v[1, pl.ds(o, L)]
            pz = pos_v[2, pl.ds(o, L)]
            plsc.addupdate_scatter(acc_v, [bidx], ones, mask=cond)
            plsc.addupdate_scatter(acc_v, [bidx + b], px, mask=cond)
            plsc.addupdate_scatter(acc_v, [bidx + 2 * b], py, mask=cond)
            plsc.addupdate_scatter(acc_v, [bidx + 3 * b], pz, mask=cond)
            return carry

        for d in dA:
            d.wait()
        lax.fori_loop(0, gA, body, 0)
        for d in dB:
            d.wait()
        lax.fori_loop(gA, groups, body, 0)

        pltpu.sync_copy(acc_v, sp.at[s])
        plsc.subcore_barrier()

        @pl.when(s < nstripes)
        def _():
            col0 = s * stripe
            pltpu.sync_copy(sp.at[:, pl.ds(col0, stripe)], red_v)
            for k in range(stripe // L):
                t = red_v[0, pl.ds(k * L, L)]
                for i in range(1, NS):
                    t = t + red_v[i, pl.ds(k * L, L)]
                str_v[pl.ds(k * L, L)] = t
            pltpu.sync_copy(str_v, out_hbm.at[c, pl.ds(col0, stripe)])

    return sc_kernel(h_t, pos_t, batch_idx)


def _tc_combine(partials, h_t, pos_t, idx_tail, n, d, b, tail):
    blk = (n - tail) // TILE

    def body(p_ref, ht_ref, pt3_ref, it_ref, pt_ref, pd_ref):
        s = jnp.sum(p_ref[...], axis=0, keepdims=True)

        ht = ht_ref[...]
        c0 = ht[3:4, :]
        c1 = ht[4:5, :]
        c2 = ht[5:6, :]
        c3 = ht[6:7, :]
        w = ((c1 > c0) & (c1 >= c2) & (c1 >= c3)).astype(jnp.float32)
        pos3 = pt3_ref[...]
        vals = jnp.concatenate([w, pos3 * w], axis=0)
        valsk = jax.lax.slice(vals, (0, 0), (4, tail))
        seg = jax.lax.broadcasted_iota(jnp.int32, (b, tail), 0)
        oh = (seg == it_ref[...][None, :]).astype(jnp.float32)
        corr = jax.lax.dot_general(
            valsk, oh, (((1,), (1,)), ((), ())),
            precision=jax.lax.Precision.HIGHEST,
            preferred_element_type=jnp.float32,
        )

        cnt = s[:, 0:b] + corr[0:1]
        sx = s[:, b:2 * b] + corr[1:2]
        sy = s[:, 2 * b:3 * b] + corr[2:3]
        sz = s[:, 3 * b:4 * b] + corr[3:4]
        c = jnp.maximum(cnt, 1.0)
        mx, my, mz = sx / c, sy / c, sz / c
        pt = jnp.sqrt(mx * mx + my * my + mz * mz)
        pt_ref[...] = pt
        pd_ref[...] = jnp.concatenate([mx / pt, my / pt, mz / pt], axis=0)

    return pl.pallas_call(
        body,
        grid=(1,),
        in_specs=[
            pl.BlockSpec(partials.shape, lambda i: (0, 0)),
            pl.BlockSpec((d, TILE), lambda i: (0, blk)),
            pl.BlockSpec((3, TILE), lambda i: (0, blk)),
            pl.BlockSpec((tail,), lambda i: (0,)),
        ],
        out_specs=[
            pl.BlockSpec((1, b), lambda i: (0, 0)),
            pl.BlockSpec((3, b), lambda i: (0, 0)),
        ],
        out_shape=[
            jax.ShapeDtypeStruct((1, b), jnp.float32),
            jax.ShapeDtypeStruct((3, b), jnp.float32),
        ],
    )(partials, h_t, pos_t, idx_tail)


def kernel(x_global_features, h, pos_pxpypz_at_vertex, batch_idx):
    n, d = h.shape
    b = x_global_features.shape[0]
    tail = n % TILE
    h_t = h.T
    pos_t = pos_pxpypz_at_vertex.T
    idx_tail = lax.slice(batch_idx, (n - tail,), (n,))
    partials = _sc_partials(h_t, pos_t, batch_idx, n, d, b)
    pt, pd = _tc_combine(partials, h_t, pos_t, idx_tail, n, d, b, tail)
    return pt.reshape(b), pd.T
